# initial kernel scaffold (unmeasured)
import jax
import jax.numpy as jnp
from jax import lax
from jax.experimental import pallas as pl
from jax.experimental.pallas import tpu as pltpu

N_DEV = 4
E_LOC = 4
E = N_DEV * E_LOC
N_TOK = 1024
D = 512
H = 1024


def kernel(x, router_W, route_idx, expert_W):
    ew_bf16 = expert_W.astype(jnp.bfloat16)

    def body(x_ref, rw_ref, idx_ref, ew_ref, out_ref, comm_ref, send_sems, recv_sems):
        my = lax.axis_index("i")
        left = lax.rem(my - 1 + N_DEV, N_DEV)
        right = lax.rem(my + 1, N_DEV)

        barrier_sem = pltpu.get_barrier_semaphore()
        for nbr in (left, right):
            pl.semaphore_signal(
                barrier_sem, inc=1,
                device_id=(nbr,), device_id_type=pl.DeviceIdType.MESH,
            )
        pl.semaphore_wait(barrier_sem, 2)

        xf = x_ref[:, :]
        scores = jnp.dot(xf, rw_ref[:, :], preferred_element_type=jnp.float32)
        m = jnp.max(scores, axis=1, keepdims=True)
        p = jnp.exp(scores - m)
        p = p / jnp.sum(p, axis=1, keepdims=True)
        e0 = idx_ref[:, 0:1]
        e1 = idx_ref[:, 1:2]
        lanes = lax.broadcasted_iota(jnp.int32, (N_TOK, E), 1)
        m0 = lanes == e0
        m1 = lanes == e1
        g0 = jnp.sum(jnp.where(m0, p, 0.0), axis=1, keepdims=True)
        g1 = jnp.sum(jnp.where(m1, p, 0.0), axis=1, keepdims=True)
        gs = g0 + g1
        gate = jnp.where(m0, g0 / gs, 0.0) + jnp.where(m1, g1 / gs, 0.0)

        def compute_block(src_chip, wblk):
            rows = lax.broadcasted_iota(jnp.int32, (E, E_LOC), 0)
            cols = lax.broadcasted_iota(jnp.int32, (E, E_LOC), 1)
            sel = (rows == src_chip * E_LOC + cols).astype(jnp.float32)
            gblk = jnp.dot(gate, sel, preferred_element_type=jnp.float32)
            acc = out_ref[:, :]
            for j in range(E_LOC):
                xg = (xf * gblk[:, j:j + 1]).astype(jnp.bfloat16)
                acc = acc + jnp.dot(xg, wblk[j], preferred_element_type=jnp.float32)
            out_ref[:, :] = acc

        out_ref[:, :] = jnp.zeros((N_TOK, H), jnp.float32)

        for h in range(N_DEV - 1):
            src = ew_ref if h == 0 else comm_ref.at[h - 1]
            rdma = pltpu.make_async_remote_copy(
                src_ref=src,
                dst_ref=comm_ref.at[h],
                send_sem=send_sems.at[h],
                recv_sem=recv_sems.at[h],
                device_id=(right,),
                device_id_type=pl.DeviceIdType.MESH,
            )
            rdma.start()
            src_chip = lax.rem(my - h + N_DEV, N_DEV)
            wblk = ew_ref[:, :, :] if h == 0 else comm_ref[h - 1]
            compute_block(src_chip, wblk)
            rdma.wait()

        compute_block(lax.rem(my - 3 + N_DEV, N_DEV), comm_ref[2])

    return pl.pallas_call(
        body,
        out_shape=jax.ShapeDtypeStruct((N_TOK, H), jnp.float32),
        in_specs=[
            pl.BlockSpec(memory_space=pltpu.VMEM),
            pl.BlockSpec(memory_space=pltpu.VMEM),
            pl.BlockSpec(memory_space=pltpu.VMEM),
            pl.BlockSpec(memory_space=pltpu.VMEM),
        ],
        out_specs=pl.BlockSpec(memory_space=pltpu.VMEM),
        scratch_shapes=[
            pltpu.VMEM((N_DEV - 1, E_LOC, D, H), jnp.bfloat16),
            pltpu.SemaphoreType.DMA((N_DEV - 1,)),
            pltpu.SemaphoreType.DMA((N_DEV - 1,)),
        ],
        compiler_params=pltpu.CompilerParams(collective_id=0),
    )(x, router_W, route_idx, ew_bf16)


# baseline (device time: 157736 ns/iter reference)
import jax
import jax.numpy as jnp
from jax import lax
from jax.experimental import pallas as pl
from jax.experimental.pallas import tpu as pltpu

N_DEV = 4
E_LOC = 4
E = N_DEV * E_LOC
N_TOK = 1024
D = 512
H = 1024


def kernel(x, router_W, route_idx, expert_W):
    ew_bf16 = expert_W.astype(jnp.bfloat16)

    def body(x_ref, rw_ref, idx_ref, ew_ref, out_ref, comm_ref, send_sems, recv_sems):
        my = lax.axis_index("i")
        left = lax.rem(my - 1 + N_DEV, N_DEV)
        right = lax.rem(my + 1, N_DEV)

        barrier_sem = pltpu.get_barrier_semaphore()
        for nbr in (left, right):
            pl.semaphore_signal(
                barrier_sem, inc=1,
                device_id=(nbr,), device_id_type=pl.DeviceIdType.MESH,
            )
        pl.semaphore_wait(barrier_sem, 2)

        xf = x_ref[:, :]
        scores = jnp.dot(xf, rw_ref[:, :], preferred_element_type=jnp.float32)
        m = jnp.max(scores, axis=1, keepdims=True)
        p = jnp.exp(scores - m)
        p = p / jnp.sum(p, axis=1, keepdims=True)
        e0 = idx_ref[:, 0:1]
        e1 = idx_ref[:, 1:2]
        lanes = lax.broadcasted_iota(jnp.int32, (N_TOK, E), 1)
        m0 = lanes == e0
        m1 = lanes == e1
        g0 = jnp.sum(jnp.where(m0, p, 0.0), axis=1, keepdims=True)
        g1 = jnp.sum(jnp.where(m1, p, 0.0), axis=1, keepdims=True)
        gs = g0 + g1
        gate = jnp.where(m0, g0 / gs, 0.0) + jnp.where(m1, g1 / gs, 0.0)

        def compute_block(src_chip, wblk):
            rows = lax.broadcasted_iota(jnp.int32, (E, E_LOC), 0)
            cols = lax.broadcasted_iota(jnp.int32, (E, E_LOC), 1)
            sel = (rows == src_chip * E_LOC + cols).astype(jnp.float32)
            gblk = jnp.dot(gate, sel, preferred_element_type=jnp.float32)
            for j in range(E_LOC):
                xg = (xf * gblk[:, j:j + 1]).astype(jnp.bfloat16)
                out_ref[:, :] = out_ref[:, :] + jnp.dot(
                    xg, wblk[j], preferred_element_type=jnp.float32
                )

        out_ref[:, :] = jnp.zeros((N_TOK, H), jnp.float32)

        for h in range(N_DEV - 1):
            src = ew_ref if h == 0 else comm_ref.at[h - 1]
            rdma = pltpu.make_async_remote_copy(
                src_ref=src,
                dst_ref=comm_ref.at[h],
                send_sem=send_sems.at[h],
                recv_sem=recv_sems.at[h],
                device_id=(right,),
                device_id_type=pl.DeviceIdType.MESH,
            )
            rdma.start()
            src_chip = lax.rem(my - h + N_DEV, N_DEV)
            wblk = ew_ref[:, :, :] if h == 0 else comm_ref[h - 1]
            compute_block(src_chip, wblk)
            rdma.wait()

        compute_block(lax.rem(my - 3 + N_DEV, N_DEV), comm_ref[2])

    return pl.pallas_call(
        body,
        out_shape=jax.ShapeDtypeStruct((N_TOK, H), jnp.float32),
        in_specs=[
            pl.BlockSpec(memory_space=pltpu.VMEM),
            pl.BlockSpec(memory_space=pltpu.VMEM),
            pl.BlockSpec(memory_space=pltpu.VMEM),
            pl.BlockSpec(memory_space=pltpu.VMEM),
        ],
        out_specs=pl.BlockSpec(memory_space=pltpu.VMEM),
        scratch_shapes=[
            pltpu.VMEM((N_DEV - 1, E_LOC, D, H), jnp.bfloat16),
            pltpu.SemaphoreType.DMA((N_DEV - 1,)),
            pltpu.SemaphoreType.DMA((N_DEV - 1,)),
        ],
        compiler_params=pltpu.CompilerParams(collective_id=0),
    )(x, router_W, route_idx, ew_bf16)


# device time: 90086 ns/iter; 1.7509x vs baseline; 1.7509x over previous
import jax
import jax.numpy as jnp
from jax import lax
from jax.experimental import pallas as pl
from jax.experimental.pallas import tpu as pltpu

N_DEV = 4
E_LOC = 4
E_HALF = 2
E = N_DEV * E_LOC
N_TOK = 1024
D = 512
H = 1024


def kernel(x, router_W, route_idx, expert_W):
    ew_bf16 = expert_W.astype(jnp.bfloat16)
    ew_cw = ew_bf16[:E_HALF]
    ew_ccw = ew_bf16[E_HALF:]

    def body(x_ref, rw_ref, idx_ref, cw_ref, ccw_ref, out_ref,
             comm_cw, comm_ccw, send_cw, recv_cw, send_ccw, recv_ccw):
        my = lax.axis_index("i")
        left = lax.rem(my - 1 + N_DEV, N_DEV)
        right = lax.rem(my + 1, N_DEV)

        barrier_sem = pltpu.get_barrier_semaphore()
        for nbr in (left, right):
            pl.semaphore_signal(
                barrier_sem, inc=1,
                device_id=(nbr,), device_id_type=pl.DeviceIdType.MESH,
            )
        pl.semaphore_wait(barrier_sem, 2)

        xf = x_ref[:, :]
        scores = jnp.dot(xf, rw_ref[:, :], preferred_element_type=jnp.float32)
        m = jnp.max(scores, axis=1, keepdims=True)
        p = jnp.exp(scores - m)
        p = p / jnp.sum(p, axis=1, keepdims=True)
        e0 = idx_ref[:, 0:1]
        e1 = idx_ref[:, 1:2]
        lanes = lax.broadcasted_iota(jnp.int32, (N_TOK, E), 1)
        m0 = lanes == e0
        m1 = lanes == e1
        g0 = jnp.sum(jnp.where(m0, p, 0.0), axis=1, keepdims=True)
        g1 = jnp.sum(jnp.where(m1, p, 0.0), axis=1, keepdims=True)
        gs = g0 + g1
        gate = jnp.where(m0, g0 / gs, 0.0) + jnp.where(m1, g1 / gs, 0.0)

        def compute_experts(src_chip, off, wblk):
            rows = lax.broadcasted_iota(jnp.int32, (E, E_HALF), 0)
            cols = lax.broadcasted_iota(jnp.int32, (E, E_HALF), 1)
            sel = (rows == src_chip * E_LOC + off + cols).astype(jnp.float32)
            gblk = jnp.dot(gate, sel, preferred_element_type=jnp.float32)
            for j in range(E_HALF):
                xg = (xf * gblk[:, j:j + 1]).astype(jnp.bfloat16)
                out_ref[:, :] = out_ref[:, :] + jnp.dot(
                    xg, wblk[j], preferred_element_type=jnp.float32
                )

        out_ref[:, :] = jnp.zeros((N_TOK, H), jnp.float32)

        for h in range(N_DEV - 1):
            rd_cw = pltpu.make_async_remote_copy(
                src_ref=cw_ref if h == 0 else comm_cw.at[h - 1],
                dst_ref=comm_cw.at[h],
                send_sem=send_cw.at[h],
                recv_sem=recv_cw.at[h],
                device_id=(right,),
                device_id_type=pl.DeviceIdType.MESH,
            )
            rd_ccw = pltpu.make_async_remote_copy(
                src_ref=ccw_ref if h == 0 else comm_ccw.at[h - 1],
                dst_ref=comm_ccw.at[h],
                send_sem=send_ccw.at[h],
                recv_sem=recv_ccw.at[h],
                device_id=(left,),
                device_id_type=pl.DeviceIdType.MESH,
            )
            rd_cw.start()
            rd_ccw.start()
            if h == 0:
                compute_experts(my, 0, cw_ref[:, :, :])
                compute_experts(my, E_HALF, ccw_ref[:, :, :])
            else:
                compute_experts(lax.rem(my - h + N_DEV, N_DEV), 0, comm_cw[h - 1])
                compute_experts(lax.rem(my + h, N_DEV), E_HALF, comm_ccw[h - 1])
            rd_cw.wait()
            rd_ccw.wait()

        compute_experts(lax.rem(my + 1, N_DEV), 0, comm_cw[2])
        compute_experts(lax.rem(my - 1 + N_DEV, N_DEV), E_HALF, comm_ccw[2])

    return pl.pallas_call(
        body,
        out_shape=jax.ShapeDtypeStruct((N_TOK, H), jnp.float32),
        in_specs=[
            pl.BlockSpec(memory_space=pltpu.VMEM),
            pl.BlockSpec(memory_space=pltpu.VMEM),
            pl.BlockSpec(memory_space=pltpu.VMEM),
            pl.BlockSpec(memory_space=pltpu.VMEM),
            pl.BlockSpec(memory_space=pltpu.VMEM),
        ],
        out_specs=pl.BlockSpec(memory_space=pltpu.VMEM),
        scratch_shapes=[
            pltpu.VMEM((N_DEV - 1, E_HALF, D, H), jnp.bfloat16),
            pltpu.VMEM((N_DEV - 1, E_HALF, D, H), jnp.bfloat16),
            pltpu.SemaphoreType.DMA((N_DEV - 1,)),
            pltpu.SemaphoreType.DMA((N_DEV - 1,)),
            pltpu.SemaphoreType.DMA((N_DEV - 1,)),
            pltpu.SemaphoreType.DMA((N_DEV - 1,)),
        ],
        compiler_params=pltpu.CompilerParams(collective_id=0),
    )(x, router_W, route_idx, ew_cw, ew_ccw)


# device time: 84388 ns/iter; 1.8692x vs baseline; 1.0675x over previous
import jax
import jax.numpy as jnp
from jax import lax
from jax.experimental import pallas as pl
from jax.experimental.pallas import tpu as pltpu

N_DEV = 4
E_LOC = 4
E_HALF = 2
E = N_DEV * E_LOC
N_TOK = 1024
D = 512
H = 1024


def kernel(x, router_W, route_idx, expert_W):
    ew_bf16 = expert_W.astype(jnp.bfloat16)
    ew_cw = ew_bf16[:E_HALF]
    ew_ccw = ew_bf16[E_HALF:]

    def body(x_ref, rw_ref, idx_ref, cw_ref, ccw_ref, out_ref,
             comm_cw, comm_ccw, send_cw, recv_cw, send_ccw, recv_ccw):
        my = lax.axis_index("i")
        left = lax.rem(my - 1 + N_DEV, N_DEV)
        right = lax.rem(my + 1, N_DEV)

        barrier_sem = pltpu.get_barrier_semaphore()
        for nbr in (left, right):
            pl.semaphore_signal(
                barrier_sem, inc=1,
                device_id=(nbr,), device_id_type=pl.DeviceIdType.MESH,
            )
        pl.semaphore_wait(barrier_sem, 2)

        xf = x_ref[:, :]
        scores = jnp.dot(xf, rw_ref[:, :], preferred_element_type=jnp.float32)
        m = jnp.max(scores, axis=1, keepdims=True)
        p = jnp.exp(scores - m)
        p = p / jnp.sum(p, axis=1, keepdims=True)
        e0 = idx_ref[:, 0:1]
        e1 = idx_ref[:, 1:2]
        lanes = lax.broadcasted_iota(jnp.int32, (N_TOK, E), 1)
        m0 = lanes == e0
        m1 = lanes == e1
        g0 = jnp.sum(jnp.where(m0, p, 0.0), axis=1, keepdims=True)
        g1 = jnp.sum(jnp.where(m1, p, 0.0), axis=1, keepdims=True)
        gs = g0 + g1
        gate = jnp.where(m0, g0 / gs, 0.0) + jnp.where(m1, g1 / gs, 0.0)

        def compute_one(src_chip, off, w2d):
            rows = lax.broadcasted_iota(jnp.int32, (E, 1), 0)
            sel = (rows == src_chip * E_LOC + off).astype(jnp.float32)
            gcol = jnp.dot(gate, sel, preferred_element_type=jnp.float32)
            xg = (xf * gcol).astype(jnp.bfloat16)
            out_ref[:, :] = out_ref[:, :] + jnp.dot(
                xg, w2d, preferred_element_type=jnp.float32
            )

        out_ref[:, :] = jnp.zeros((N_TOK, H), jnp.float32)

        def make_rdma(h, j, ccw):
            src_local = (ccw_ref if ccw else cw_ref).at[j]
            comm = comm_ccw if ccw else comm_cw
            return pltpu.make_async_remote_copy(
                src_ref=src_local if h == 0 else comm.at[h - 1, j],
                dst_ref=comm.at[h, j],
                send_sem=(send_ccw if ccw else send_cw).at[h, j],
                recv_sem=(recv_ccw if ccw else recv_cw).at[h, j],
                device_id=(left if ccw else right,),
                device_id_type=pl.DeviceIdType.MESH,
            )

        rdmas = {}
        for j in range(E_HALF):
            for ccw in (False, True):
                rdmas[(0, j, ccw)] = make_rdma(0, j, ccw)
                rdmas[(0, j, ccw)].start()
        for off in range(E_LOC):
            compute_one(my, off, (cw_ref if off < E_HALF else ccw_ref)[off % E_HALF])

        for h in (1, 2):
            for j in range(E_HALF):
                for ccw in (False, True):
                    rdmas[(h - 1, j, ccw)].wait_recv()
                    rdmas[(h, j, ccw)] = make_rdma(h, j, ccw)
                    rdmas[(h, j, ccw)].start()
                compute_one(lax.rem(my - h + N_DEV, N_DEV), j, comm_cw[h - 1, j])
                compute_one(lax.rem(my + h, N_DEV), E_HALF + j, comm_ccw[h - 1, j])

        for j in range(E_HALF):
            rdmas[(2, j, False)].wait_recv()
            compute_one(lax.rem(my + 1, N_DEV), j, comm_cw[2, j])
            rdmas[(2, j, True)].wait_recv()
            compute_one(lax.rem(my - 1 + N_DEV, N_DEV), E_HALF + j, comm_ccw[2, j])

        for rd in rdmas.values():
            rd.wait_send()

    return pl.pallas_call(
        body,
        out_shape=jax.ShapeDtypeStruct((N_TOK, H), jnp.float32),
        in_specs=[
            pl.BlockSpec(memory_space=pltpu.VMEM),
            pl.BlockSpec(memory_space=pltpu.VMEM),
            pl.BlockSpec(memory_space=pltpu.VMEM),
            pl.BlockSpec(memory_space=pltpu.VMEM),
            pl.BlockSpec(memory_space=pltpu.VMEM),
        ],
        out_specs=pl.BlockSpec(memory_space=pltpu.VMEM),
        scratch_shapes=[
            pltpu.VMEM((N_DEV - 1, E_HALF, D, H), jnp.bfloat16),
            pltpu.VMEM((N_DEV - 1, E_HALF, D, H), jnp.bfloat16),
            pltpu.SemaphoreType.DMA((N_DEV - 1, E_HALF)),
            pltpu.SemaphoreType.DMA((N_DEV - 1, E_HALF)),
            pltpu.SemaphoreType.DMA((N_DEV - 1, E_HALF)),
            pltpu.SemaphoreType.DMA((N_DEV - 1, E_HALF)),
        ],
        compiler_params=pltpu.CompilerParams(collective_id=0),
    )(x, router_W, route_idx, ew_cw, ew_ccw)


# device time: 82656 ns/iter; 1.9083x vs baseline; 1.0210x over previous
import jax
import jax.numpy as jnp
from jax import lax
from jax.experimental import pallas as pl
from jax.experimental.pallas import tpu as pltpu

N_DEV = 4
E_LOC = 4
E_HALF = 2
E = N_DEV * E_LOC
N_TOK = 1024
D = 512
H = 1024


def kernel(x, router_W, route_idx, expert_W):
    ew_bf16 = expert_W.astype(jnp.bfloat16)
    ew_cw = ew_bf16[:E_HALF]
    ew_ccw = ew_bf16[E_HALF:]

    def body(x_ref, rw_ref, idx_ref, cw_ref, ccw_ref, out_ref,
             comm_cw, comm_ccw, send_cw, recv_cw, send_ccw, recv_ccw):
        my = lax.axis_index("i")
        left = lax.rem(my - 1 + N_DEV, N_DEV)
        right = lax.rem(my + 1, N_DEV)

        barrier_sem = pltpu.get_barrier_semaphore()
        for nbr in (left, right):
            pl.semaphore_signal(
                barrier_sem, inc=1,
                device_id=(nbr,), device_id_type=pl.DeviceIdType.MESH,
            )
        pl.semaphore_wait(barrier_sem, 2)

        def make_rdma(h, j, ccw):
            src_local = (ccw_ref if ccw else cw_ref).at[j]
            comm = comm_ccw if ccw else comm_cw
            return pltpu.make_async_remote_copy(
                src_ref=src_local if h == 0 else comm.at[h - 1, j],
                dst_ref=comm.at[h, j],
                send_sem=(send_ccw if ccw else send_cw).at[h, j],
                recv_sem=(recv_ccw if ccw else recv_cw).at[h, j],
                device_id=(left if ccw else right,),
                device_id_type=pl.DeviceIdType.MESH,
            )

        rdmas = {}
        for j in range(E_HALF):
            for ccw in (False, True):
                rdmas[(0, j, ccw)] = make_rdma(0, j, ccw)
                rdmas[(0, j, ccw)].start()

        xf = x_ref[:, :]
        scores = jnp.dot(xf, rw_ref[:, :], preferred_element_type=jnp.float32)
        m = jnp.max(scores, axis=1, keepdims=True)
        p = jnp.exp(scores - m)
        p = p / jnp.sum(p, axis=1, keepdims=True)
        e0 = idx_ref[:, 0:1]
        e1 = idx_ref[:, 1:2]
        lanes = lax.broadcasted_iota(jnp.int32, (N_TOK, E), 1)
        m0 = lanes == e0
        m1 = lanes == e1
        g0 = jnp.sum(jnp.where(m0, p, 0.0), axis=1, keepdims=True)
        g1 = jnp.sum(jnp.where(m1, p, 0.0), axis=1, keepdims=True)
        gs = g0 + g1
        gate = jnp.where(m0, g0 / gs, 0.0) + jnp.where(m1, g1 / gs, 0.0)

        def compute_one(src_chip, off, w2d):
            rows = lax.broadcasted_iota(jnp.int32, (E, 1), 0)
            sel = (rows == src_chip * E_LOC + off).astype(jnp.float32)
            gcol = jnp.dot(gate, sel, preferred_element_type=jnp.float32)
            xg = (xf * gcol).astype(jnp.bfloat16)
            out_ref[:, :] = out_ref[:, :] + jnp.dot(
                xg, w2d, preferred_element_type=jnp.float32
            )

        out_ref[:, :] = jnp.zeros((N_TOK, H), jnp.float32)

        for off in range(E_LOC):
            compute_one(my, off, (cw_ref if off < E_HALF else ccw_ref)[off % E_HALF])

        for h in (1, 2):
            for j in range(E_HALF):
                for ccw in (False, True):
                    rdmas[(h - 1, j, ccw)].wait_recv()
                    rdmas[(h, j, ccw)] = make_rdma(h, j, ccw)
                    rdmas[(h, j, ccw)].start()
                compute_one(lax.rem(my - h + N_DEV, N_DEV), j, comm_cw[h - 1, j])
                compute_one(lax.rem(my + h, N_DEV), E_HALF + j, comm_ccw[h - 1, j])

        for j in range(E_HALF):
            rdmas[(2, j, False)].wait_recv()
            compute_one(lax.rem(my + 1, N_DEV), j, comm_cw[2, j])
            rdmas[(2, j, True)].wait_recv()
            compute_one(lax.rem(my - 1 + N_DEV, N_DEV), E_HALF + j, comm_ccw[2, j])

        for rd in rdmas.values():
            rd.wait_send()

    return pl.pallas_call(
        body,
        out_shape=jax.ShapeDtypeStruct((N_TOK, H), jnp.float32),
        in_specs=[
            pl.BlockSpec(memory_space=pltpu.VMEM),
            pl.BlockSpec(memory_space=pltpu.VMEM),
            pl.BlockSpec(memory_space=pltpu.VMEM),
            pl.BlockSpec(memory_space=pltpu.VMEM),
            pl.BlockSpec(memory_space=pltpu.VMEM),
        ],
        out_specs=pl.BlockSpec(memory_space=pltpu.VMEM),
        scratch_shapes=[
            pltpu.VMEM((N_DEV - 1, E_HALF, D, H), jnp.bfloat16),
            pltpu.VMEM((N_DEV - 1, E_HALF, D, H), jnp.bfloat16),
            pltpu.SemaphoreType.DMA((N_DEV - 1, E_HALF)),
            pltpu.SemaphoreType.DMA((N_DEV - 1, E_HALF)),
            pltpu.SemaphoreType.DMA((N_DEV - 1, E_HALF)),
            pltpu.SemaphoreType.DMA((N_DEV - 1, E_HALF)),
        ],
        compiler_params=pltpu.CompilerParams(collective_id=0),
    )(x, router_W, route_idx, ew_cw, ew_ccw)


# device time: 77052 ns/iter; 2.0471x vs baseline; 1.0727x over previous
import jax
import jax.numpy as jnp
from jax import lax
from jax.experimental import pallas as pl
from jax.experimental.pallas import tpu as pltpu

N_DEV = 4
E_LOC = 4
E = N_DEV * E_LOC
N_TOK = 1024
HALF = N_TOK // 2
D = 512
H = 1024

KX, KG, KRS = 0, 1, 2


def kernel(x, router_W, route_idx, expert_W):
    ew_bf16 = expert_W.astype(jnp.bfloat16)
    x_bf16 = x.astype(jnp.bfloat16)
    xa = x_bf16[:HALF]
    xb = x_bf16[HALF:]

    def body(x_ref, rw_ref, idx_ref, ew_ref, xa_ref, xb_ref, out_ref,
             agx_cw, agx_ccw, agg_cw, agg_ccw, gsrc_cw, gsrc_ccw,
             rs_start_cw, rs_start_ccw, rs_recv_cw, rs_recv_ccw,
             pacc_cw, pacc_ccw, send_cw, recv_cw, send_ccw, recv_ccw):
        my = lax.axis_index("i")
        left = lax.rem(my - 1 + N_DEV, N_DEV)
        right = lax.rem(my + 1, N_DEV)

        barrier_sem = pltpu.get_barrier_semaphore()
        for nbr in (left, right):
            pl.semaphore_signal(
                barrier_sem, inc=1,
                device_id=(nbr,), device_id_type=pl.DeviceIdType.MESH,
            )
        pl.semaphore_wait(barrier_sem, 2)

        def make_copy(kind, h, ccw, src, dst):
            return pltpu.make_async_remote_copy(
                src_ref=src,
                dst_ref=dst,
                send_sem=(send_ccw if ccw else send_cw).at[kind, h],
                recv_sem=(recv_ccw if ccw else recv_cw).at[kind, h],
                device_id=(left if ccw else right,),
                device_id_type=pl.DeviceIdType.MESH,
            )

        rdmas = {}

        def start(kind, h, ccw, src, dst):
            rd = make_copy(kind, h, ccw, src, dst)
            rdmas[(kind, h, ccw)] = rd
            rd.start()

        start(KX, 0, False, xa_ref, agx_cw.at[0])
        start(KX, 0, True, xb_ref, agx_ccw.at[0])

        xf = x_ref[:, :]
        scores = jnp.dot(xf, rw_ref[:, :], preferred_element_type=jnp.float32)
        m = jnp.max(scores, axis=1, keepdims=True)
        p = jnp.exp(scores - m)
        p = p / jnp.sum(p, axis=1, keepdims=True)
        e0 = idx_ref[:, 0:1]
        e1 = idx_ref[:, 1:2]
        lanes = lax.broadcasted_iota(jnp.int32, (N_TOK, E), 1)
        m0 = lanes == e0
        m1 = lanes == e1
        g0 = jnp.sum(jnp.where(m0, p, 0.0), axis=1, keepdims=True)
        g1 = jnp.sum(jnp.where(m1, p, 0.0), axis=1, keepdims=True)
        gs = g0 + g1
        gate = jnp.where(m0, g0 / gs, 0.0) + jnp.where(m1, g1 / gs, 0.0)

        gsrc_cw[:, :] = gate[:HALF].astype(jnp.bfloat16)
        gsrc_ccw[:, :] = gate[HALF:].astype(jnp.bfloat16)
        start(KG, 0, False, gsrc_cw, agg_cw.at[0])
        start(KG, 0, True, gsrc_ccw, agg_ccw.at[0])

        def onehot(col):
            rows = lax.broadcasted_iota(jnp.int32, (E, 1), 0)
            return (rows == col).astype(jnp.float32)

        for j in range(E_LOC):
            gcol = jnp.dot(gate, onehot(my * E_LOC + j),
                           preferred_element_type=jnp.float32)
            xg = (xf * gcol).astype(jnp.bfloat16)
            d = jnp.dot(xg, ew_ref[j], preferred_element_type=jnp.float32)
            if j == 0:
                out_ref[:, :] = d
            else:
                out_ref[:, :] = out_ref[:, :] + d

        def compute_pacc(ccw, h):
            agx = agx_ccw if ccw else agx_cw
            agg = agg_ccw if ccw else agg_cw
            pacc = pacc_ccw if ccw else pacc_cw
            xr = agx[h]
            gr = agg[h].astype(jnp.float32)
            for j in range(E_LOC):
                gcol = jnp.dot(gr, onehot(my * E_LOC + j),
                               preferred_element_type=jnp.float32)
                xg = (xr * gcol).astype(jnp.bfloat16)
                d = jnp.dot(xg, ew_ref[j], preferred_element_type=jnp.float32)
                if j == 0:
                    pacc[:, :] = d
                else:
                    pacc[:, :] = pacc[:, :] + d

        for h in range(N_DEV - 1):
            for ccw in (False, True):
                agx = agx_ccw if ccw else agx_cw
                agg = agg_ccw if ccw else agg_cw
                pacc = pacc_ccw if ccw else pacc_cw
                rs_start = rs_start_ccw if ccw else rs_start_cw
                rs_recv = rs_recv_ccw if ccw else rs_recv_cw
                rdmas[(KX, h, ccw)].wait_recv()
                rdmas[(KG, h, ccw)].wait_recv()
                if h < N_DEV - 2:
                    start(KX, h + 1, ccw, agx.at[h], agx.at[h + 1])
                    start(KG, h + 1, ccw, agg.at[h], agg.at[h + 1])
                compute_pacc(ccw, h)
                if h == 0:
                    rs_start[:, :] = pacc[:, :].astype(jnp.bfloat16)
                    start(KRS, 0, ccw, rs_start, rs_recv.at[0])
                else:
                    rdmas[(KRS, h - 1, ccw)].wait_recv()
                    rs_recv[h - 1, :, :] = (
                        rs_recv[h - 1].astype(jnp.float32) + pacc[:, :]
                    ).astype(jnp.bfloat16)
                    start(KRS, h, ccw, rs_recv.at[h - 1], rs_recv.at[h])

        rdmas[(KRS, 2, False)].wait_recv()
        out_ref[:HALF, :] = out_ref[:HALF, :] + rs_recv_cw[2].astype(jnp.float32)
        rdmas[(KRS, 2, True)].wait_recv()
        out_ref[HALF:, :] = out_ref[HALF:, :] + rs_recv_ccw[2].astype(jnp.float32)

        for rd in rdmas.values():
            rd.wait_send()

    return pl.pallas_call(
        body,
        out_shape=jax.ShapeDtypeStruct((N_TOK, H), jnp.float32),
        in_specs=[pl.BlockSpec(memory_space=pltpu.VMEM)] * 6,
        out_specs=pl.BlockSpec(memory_space=pltpu.VMEM),
        scratch_shapes=[
            pltpu.VMEM((N_DEV - 1, HALF, D), jnp.bfloat16),
            pltpu.VMEM((N_DEV - 1, HALF, D), jnp.bfloat16),
            pltpu.VMEM((N_DEV - 1, HALF, E), jnp.bfloat16),
            pltpu.VMEM((N_DEV - 1, HALF, E), jnp.bfloat16),
            pltpu.VMEM((HALF, E), jnp.bfloat16),
            pltpu.VMEM((HALF, E), jnp.bfloat16),
            pltpu.VMEM((HALF, H), jnp.bfloat16),
            pltpu.VMEM((HALF, H), jnp.bfloat16),
            pltpu.VMEM((N_DEV - 1, HALF, H), jnp.bfloat16),
            pltpu.VMEM((N_DEV - 1, HALF, H), jnp.bfloat16),
            pltpu.VMEM((HALF, H), jnp.float32),
            pltpu.VMEM((HALF, H), jnp.float32),
            pltpu.SemaphoreType.DMA((3, N_DEV - 1)),
            pltpu.SemaphoreType.DMA((3, N_DEV - 1)),
            pltpu.SemaphoreType.DMA((3, N_DEV - 1)),
            pltpu.SemaphoreType.DMA((3, N_DEV - 1)),
        ],
        compiler_params=pltpu.CompilerParams(collective_id=0),
    )(x, router_W, route_idx, ew_bf16, xa, xb)
